# baseline (speedup 1.0000x reference)
import jax
import jax.numpy as jnp
from jax.experimental import pallas as pl
from jax.experimental.pallas import tpu as pltpu

BT = 512

C0, H0, W0 = 3, 32, 32
C1, K1, P1 = 10, 3, 3
H1, W1 = 30, 30
HP, WP = 10, 10
C2, K2, P2 = 5, 3, 4
H2, W2 = 8, 8
HQ, WQ = 2, 2
NF = C2 * HQ * WQ
NOUT = 2


def _conv1_operator(w1):
    w1r = w1.reshape(C1, C0, K1, K1)
    w1p = jnp.pad(w1r, ((0, 0), (0, 0), (0, 0), (0, 30)))
    band = jnp.broadcast_to(w1p[:, :, :, None, :],
                            (C1, C0, K1, W1, 33)).reshape(C1, C0, K1, 990)
    band = band[:, :, :, :960].reshape(C1, C0, K1, W1, 32)
    rows = [jnp.pad(band, ((0, 0), (0, 0), (r, 2 - r), (0, 0), (0, 0)))
            for r in range(3)]
    t = jnp.stack(rows, axis=1)
    t = t.reshape(C1, 3, C0, 5, WP, 3, 32)
    t = t.transpose(0, 1, 5, 4, 3, 2, 6)
    t = t.reshape(C1, 3, 30, 5, C0, 32)
    t = jnp.pad(t, ((0, 0), (0, 0), (0, 2), (0, 0), (0, 0), (0, 0)))
    return t.reshape(C1 * 96, 5 * C0 * 32)


def _conv2_operator(w2):
    w2r = w2.reshape(C2, C1, K2, K2)
    w2p = jnp.pad(w2r, ((0, 0), (0, 0), (0, 0), (0, 14)))
    band = jnp.broadcast_to(w2p[:, :, :, None, :],
                            (C2, C1, K2, H2, 17)).reshape(C2, C1, K2, 136)
    band = band[:, :, :, :128].reshape(C2, C1, K2, H2, 16)
    rows = [jnp.pad(band, ((0, 0), (0, 0), (h, 7 - h), (0, 0), (0, 0)))
            for h in range(H2)]
    t = jnp.stack(rows, axis=1)
    t = t.reshape(C2, H2, C1, HP, WQ, 4, 16)
    t = t.transpose(0, 1, 5, 4, 3, 2, 6)
    t = t.reshape(C2, H2, 8, HP, C1, 16)
    return t.reshape(C2 * 64, HP * 160)


def _hilo(a):
    ui = jax.lax.bitcast_convert_type(a, jnp.uint32)
    hi_f = jax.lax.bitcast_convert_type(ui & jnp.uint32(0xFFFF0000),
                                        jnp.float32)
    return hi_f.astype(jnp.bfloat16), (a - hi_f).astype(jnp.bfloat16)


def _cat3(a, nblk, blk):
    ah, al = _hilo(a)
    ah3 = ah.reshape(a.shape[0], nblk, blk)
    al3 = al.reshape(a.shape[0], nblk, blk)
    return jnp.stack([ah3, ah3, al3], axis=2).reshape(a.shape[0],
                                                      nblk * 3 * blk)


def _body(x_ref, a3_ref, a23_ref, o_ref,
          xc_ref, o1a_ref, o1b_ref, hsa_ref, hsb_ref, p1_ref, o2_ref):
    f32 = jnp.float32
    p1_ref[...] = jnp.zeros(p1_ref.shape, jnp.bfloat16)

    for h in range(H0):
        v = x_ref[pl.ds(h * 96, 96), :]
        vh, vl = _hilo(v)
        xc_ref[pl.ds(h * 288, 96), :] = vh
        xc_ref[pl.ds(h * 288 + 96, 96), :] = vl
        xc_ref[pl.ds(h * 288 + 192, 96), :] = vh

    for ph in range(HP):
        o1_ref = o1a_ref if ph % 2 == 0 else o1b_ref
        hs_ref = hsa_ref if ph % 2 == 0 else hsb_ref
        slab = xc_ref[pl.ds(3 * ph * 288, 1440), :]
        o1_ref[...] = jnp.dot(a3_ref[...], slab,
                              preferred_element_type=f32).reshape(
            C1, 3, 32, BT)
        hs_ref[...] = jnp.maximum(jnp.maximum(o1_ref[:, 0], o1_ref[:, 1]),
                                  o1_ref[:, 2])
        pooled = jnp.maximum(
            jnp.maximum(hs_ref[:, 0:WP, :], hs_ref[:, WP:2 * WP, :]),
            hs_ref[:, 2 * WP:3 * WP, :])
        ph_h, ph_l = _hilo(pooled)
        for ci in range(C1):
            p1_ref[pl.ds(ph * 480 + ci * 16, WP), :] = ph_h[ci]
            p1_ref[pl.ds(ph * 480 + 160 + ci * 16, WP), :] = ph_l[ci]
            p1_ref[pl.ds(ph * 480 + 320 + ci * 16, WP), :] = ph_h[ci]

    o2_ref[...] = jnp.dot(a23_ref[...], p1_ref[...],
                          preferred_element_type=f32).reshape(C2, H2, W2, BT)
    for qh in range(HQ):
        hm = jnp.max(o2_ref[:, 4 * qh:4 * qh + 4], axis=1)
        p2 = jnp.maximum(jnp.maximum(hm[:, 0:2], hm[:, 2:4]),
                         jnp.maximum(hm[:, 4:6], hm[:, 6:8]))
        for c in range(C2):
            o_ref[pl.ds(c * 4 + qh * 2, WQ), :] = p2[c]


@jax.jit
def kernel(x_nchw, w1, b1, w2, b2, wl, bl):
    B = x_nchw.shape[0]
    Bp = ((B + BT - 1) // BT) * BT

    a3 = _cat3(_conv1_operator(w1), 5, 96)
    a23 = _cat3(_conv2_operator(w2), HP, 160)

    s2 = w2.reshape(C2, C1, 9).sum(-1)
    b2eff = b2 + (s2 * b1[None, :]).sum(-1)
    bleff = bl + (wl.reshape(NOUT, NF)
                  * jnp.repeat(b2eff, HQ * WQ)[None, :]).sum(-1)

    xt = jnp.transpose(x_nchw, (2, 1, 3, 0)).reshape(C0 * H0 * 32, B)
    if B != Bp:
        xt = jnp.pad(xt, ((0, 0), (0, Bp - B)))

    feat = pl.pallas_call(
        _body,
        out_shape=jax.ShapeDtypeStruct((24, Bp), jnp.float32),
        grid=(Bp // BT,),
        in_specs=[
            pl.BlockSpec((C0 * H0 * 32, BT), lambda g: (0, g)),
            pl.BlockSpec((C1 * 96, 1440), lambda g: (0, 0)),
            pl.BlockSpec((C2 * 64, HP * 480), lambda g: (0, 0)),
        ],
        out_specs=pl.BlockSpec((24, BT), lambda g: (0, g)),
        scratch_shapes=[
            pltpu.VMEM((H0 * 288, BT), jnp.bfloat16),
            pltpu.VMEM((C1, 3, 32, BT), jnp.float32),
            pltpu.VMEM((C1, 3, 32, BT), jnp.float32),
            pltpu.VMEM((C1, 32, BT), jnp.float32),
            pltpu.VMEM((C1, 32, BT), jnp.float32),
            pltpu.VMEM((HP * 480, BT), jnp.bfloat16),
            pltpu.VMEM((C2, H2, W2, BT), jnp.float32),
        ],
        compiler_params=pltpu.CompilerParams(
            dimension_semantics=("parallel",)),
    )(xt, a3, a23)
    res = (wl.reshape(NOUT, NF)[:, :, None]
           * feat[None, :NF, :B]).sum(1) + bleff[:, None]
    return res.T


# speedup vs baseline: 1.0542x; 1.0542x over previous
import jax
import jax.numpy as jnp
from jax.experimental import pallas as pl
from jax.experimental.pallas import tpu as pltpu

BT = 512

C0, H0, W0 = 3, 32, 32
C1, K1, P1 = 10, 3, 3
H1, W1 = 30, 30
HP, WP = 10, 10
C2, K2, P2 = 5, 3, 4
H2, W2 = 8, 8
HQ, WQ = 2, 2
NF = C2 * HQ * WQ
NOUT = 2


def _conv1_operator(w1):
    w1r = w1.reshape(C1, C0, K1, K1)
    w1p = jnp.pad(w1r, ((0, 0), (0, 0), (0, 0), (0, 30)))
    band = jnp.broadcast_to(w1p[:, :, :, None, :],
                            (C1, C0, K1, W1, 33)).reshape(C1, C0, K1, 990)
    band = band[:, :, :, :960].reshape(C1, C0, K1, W1, 32)
    rows = [jnp.pad(band, ((0, 0), (0, 0), (r, 2 - r), (0, 0), (0, 0)))
            for r in range(3)]
    t = jnp.stack(rows, axis=1)
    t = t.reshape(C1, 3, C0, 5, WP, 3, 32)
    t = t.transpose(0, 1, 5, 4, 3, 2, 6)
    t = t.reshape(C1, 3, 30, 5, C0, 32)
    t = jnp.pad(t, ((0, 0), (0, 0), (0, 2), (0, 0), (0, 0), (0, 0)))
    return t.reshape(C1 * 96, 5 * C0 * 32)


def _conv2_operator(w2):
    w2r = w2.reshape(C2, C1, K2, K2)
    w2p = jnp.pad(w2r, ((0, 0), (0, 0), (0, 0), (0, 14)))
    band = jnp.broadcast_to(w2p[:, :, :, None, :],
                            (C2, C1, K2, H2, 17)).reshape(C2, C1, K2, 136)
    band = band[:, :, :, :128].reshape(C2, C1, K2, H2, 16)
    rows = [jnp.pad(band, ((0, 0), (0, 0), (h, 7 - h), (0, 0), (0, 0)))
            for h in range(H2)]
    t = jnp.stack(rows, axis=1)
    t = t.reshape(C2, H2, C1, HP, WQ, 4, 16)
    t = t.transpose(0, 1, 5, 4, 3, 2, 6)
    t = t.reshape(C2, H2, 8, HP, C1, 16)
    return t.reshape(C2 * 64, HP * 160)


def _hilo(a):
    ui = jax.lax.bitcast_convert_type(a, jnp.uint32)
    hi_f = jax.lax.bitcast_convert_type(ui & jnp.uint32(0xFFFF0000),
                                        jnp.float32)
    return hi_f.astype(jnp.bfloat16), (a - hi_f).astype(jnp.bfloat16)


def _dot3(ah, al, bh, bl):
    f32 = jnp.float32
    return (jnp.dot(ah, bh, preferred_element_type=f32)
            + jnp.dot(ah, bl, preferred_element_type=f32)
            + jnp.dot(al, bh, preferred_element_type=f32))


def _body(x_ref, a1h_ref, a1l_ref, a2h_ref, a2l_ref, o_ref,
          xh_ref, xl_ref, o1a_ref, o1b_ref, hsa_ref, hsb_ref,
          p1h_ref, p1l_ref, o2_ref):
    f32 = jnp.float32
    p1h_ref[...] = jnp.zeros(p1h_ref.shape, jnp.bfloat16)
    p1l_ref[...] = jnp.zeros(p1l_ref.shape, jnp.bfloat16)

    for c in range(6):
        v = x_ref[pl.ds(c * 512, 512), :]
        vh, vl = _hilo(v)
        xh_ref[pl.ds(c * 512, 512), :] = vh
        xl_ref[pl.ds(c * 512, 512), :] = vl

    for ph in range(HP):
        o1_ref = o1a_ref if ph % 2 == 0 else o1b_ref
        hs_ref = hsa_ref if ph % 2 == 0 else hsb_ref
        sh = xh_ref[pl.ds(3 * ph * 96, 480), :]
        sl = xl_ref[pl.ds(3 * ph * 96, 480), :]
        o1_ref[...] = _dot3(a1h_ref[...], a1l_ref[...], sh, sl).reshape(
            C1, 3, 32, BT)
        hs_ref[...] = jnp.maximum(jnp.maximum(o1_ref[:, 0], o1_ref[:, 1]),
                                  o1_ref[:, 2])
        pooled = jnp.maximum(
            jnp.maximum(hs_ref[:, 0:WP, :], hs_ref[:, WP:2 * WP, :]),
            hs_ref[:, 2 * WP:3 * WP, :])
        ph_h, ph_l = _hilo(pooled)
        for ci in range(C1):
            p1h_ref[pl.ds(ph * 160 + ci * 16, WP), :] = ph_h[ci]
            p1l_ref[pl.ds(ph * 160 + ci * 16, WP), :] = ph_l[ci]

    o2_ref[...] = _dot3(a2h_ref[...], a2l_ref[...],
                        p1h_ref[...], p1l_ref[...]).reshape(C2, H2, W2, BT)
    for qh in range(HQ):
        hm = jnp.max(o2_ref[:, 4 * qh:4 * qh + 4], axis=1)
        p2 = jnp.maximum(jnp.maximum(hm[:, 0:2], hm[:, 2:4]),
                         jnp.maximum(hm[:, 4:6], hm[:, 6:8]))
        for c in range(C2):
            o_ref[pl.ds(c * 4 + qh * 2, WQ), :] = p2[c]


@jax.jit
def kernel(x_nchw, w1, b1, w2, b2, wl, bl):
    B = x_nchw.shape[0]
    Bp = ((B + BT - 1) // BT) * BT

    a1h, a1l = _hilo(_conv1_operator(w1))
    a2h, a2l = _hilo(_conv2_operator(w2))

    s2 = w2.reshape(C2, C1, 9).sum(-1)
    b2eff = b2 + (s2 * b1[None, :]).sum(-1)
    bleff = bl + (wl.reshape(NOUT, NF)
                  * jnp.repeat(b2eff, HQ * WQ)[None, :]).sum(-1)

    xt = jnp.transpose(x_nchw, (2, 1, 3, 0)).reshape(C0 * H0 * 32, B)
    if B != Bp:
        xt = jnp.pad(xt, ((0, 0), (0, Bp - B)))

    feat = pl.pallas_call(
        _body,
        out_shape=jax.ShapeDtypeStruct((24, Bp), jnp.float32),
        grid=(Bp // BT,),
        in_specs=[
            pl.BlockSpec((C0 * H0 * 32, BT), lambda g: (0, g)),
            pl.BlockSpec((C1 * 96, 480), lambda g: (0, 0)),
            pl.BlockSpec((C1 * 96, 480), lambda g: (0, 0)),
            pl.BlockSpec((C2 * 64, HP * 160), lambda g: (0, 0)),
            pl.BlockSpec((C2 * 64, HP * 160), lambda g: (0, 0)),
        ],
        out_specs=pl.BlockSpec((24, BT), lambda g: (0, g)),
        scratch_shapes=[
            pltpu.VMEM((C0 * H0 * 32, BT), jnp.bfloat16),
            pltpu.VMEM((C0 * H0 * 32, BT), jnp.bfloat16),
            pltpu.VMEM((C1, 3, 32, BT), jnp.float32),
            pltpu.VMEM((C1, 3, 32, BT), jnp.float32),
            pltpu.VMEM((C1, 32, BT), jnp.float32),
            pltpu.VMEM((C1, 32, BT), jnp.float32),
            pltpu.VMEM((HP * 160, BT), jnp.bfloat16),
            pltpu.VMEM((HP * 160, BT), jnp.bfloat16),
            pltpu.VMEM((C2, H2, W2, BT), jnp.float32),
        ],
        compiler_params=pltpu.CompilerParams(
            dimension_semantics=("parallel",)),
    )(xt, a1h, a1l, a2h, a2l)
    res = (wl.reshape(NOUT, NF)[:, :, None]
           * feat[None, :NF, :B]).sum(1) + bleff[:, None]
    return res.T
